# baseline (speedup 1.0000x reference)
import functools

import jax
import jax.numpy as jnp
from jax import lax
from jax.experimental import pallas as pl
from jax.experimental.pallas import tpu as pltpu
from jax.experimental.pallas import tpu_sc as plsc

_LOG2 = 0.6931471805599453
_SQRT3 = 1.7320508075688772


def _ssp(x):
    return jnp.maximum(x, 0.0) + jnp.log1p(jnp.exp(-jnp.abs(x))) - _LOG2


def _edge_weights_body(f_ref, d_ref, rc_ref, v_ref,
                       w1_ref, b1_ref, w2_ref, b2_ref,
                       wc_ref, u0_ref, u1_ref, u2_ref):
    f = f_ref[...]
    d = d_ref[...]
    rc = rc_ref[...]
    invd = 1.0 / d
    scale = rc * invd * invd * invd
    w1 = w1_ref[0]
    b1 = b1_ref[0]
    w2 = w2_ref[0]
    b2 = b2_ref[0]
    h = _ssp(jnp.dot(f, w1, preferred_element_type=jnp.float32) + b1)
    wc_ref[...] = (jnp.dot(h, w2, preferred_element_type=jnp.float32) + b2) * scale
    us = _SQRT3 * invd
    v = v_ref[...]
    u0_ref[...] = v[:, 0:1] * us
    u1_ref[...] = v[:, 1:2] * us
    u2_ref[...] = v[:, 2:3] * us


def _node_transform_body(pe_ref, pm_ref, wte_ref, bte_ref, wtm_ref, btm_ref, out_ref):
    ye = _ssp(jnp.dot(pe_ref[...], wte_ref[...], preferred_element_type=jnp.float32) + bte_ref[...])
    ym = _ssp(jnp.dot(pm_ref[...], wtm_ref[...], preferred_element_type=jnp.float32) + btm_ref[...])
    out_ref[...] = ye + ym


def _make_sc_stage(N, E, F):
    NSUB = 16
    EPT = E // NSUB
    C = 16
    NCH = EPT // C
    NP = (N + NSUB * C - 1) // (NSUB * C) * (NSUB * C)
    NPT = NP // NSUB
    FC = F // 16
    TF = 3 * F

    mesh = plsc.VectorSubcoreMesh(core_axis_name="c", subcore_axis_name="s")

    scratch = [pltpu.VMEM_SHARED((NP, F), jnp.float32)]
    for _ in range(2):
        scratch += [
            pltpu.VMEM((C,), jnp.int32),
            pltpu.VMEM((C,), jnp.int32),
            pltpu.VMEM((C,), jnp.int32),
            pltpu.VMEM((C,), jnp.int32),
            pltpu.VMEM((C,), jnp.int32),
            pltpu.VMEM((C,), jnp.float32),
            pltpu.VMEM((C,), jnp.float32),
            pltpu.VMEM((C,), jnp.float32),
            pltpu.VMEM((C, F), jnp.float32),
            pltpu.VMEM((C, TF), jnp.float32),
            pltpu.VMEM((C, TF), jnp.float32),
            pltpu.VMEM((C, F), jnp.float32),
        ]
    scratch += [pltpu.SemaphoreType.DMA] * 6

    @functools.partial(
        pl.kernel,
        out_type=jax.ShapeDtypeStruct((2 * NP, F), jnp.float32),
        mesh=mesh,
        scratch_types=scratch,
    )
    def sc_stage(mu_hbm, wc_hbm, u0_hbm, u1_hbm, u2_hbm, idxi_hbm, idxj_hbm,
                 out_hbm, *scr):
        acc = scr[0]
        bufs = (scr[1:13], scr[13:25])
        sem_sm = scr[25:27]
        sem_g = scr[27:29]
        sem_sc = scr[29:31]
        c = lax.axis_index("c")
        s = lax.axis_index("s")
        cN = c * N
        cE = c * E
        base0 = s * EPT

        def issue_smalls(g, par):
            idxi_v, idxj_v, _, _, _, u0v, u1v, u2v, wcv, _, _, _ = bufs[par]
            sem = sem_sm[par]
            b = base0 + g * C
            pltpu.async_copy(idxi_hbm.at[pl.ds(b, C)], idxi_v, sem)
            pltpu.async_copy(idxj_hbm.at[pl.ds(b, C)], idxj_v, sem)
            pltpu.async_copy(u0_hbm.at[pl.ds(b, C)], u0v, sem)
            pltpu.async_copy(u1_hbm.at[pl.ds(b, C)], u1v, sem)
            pltpu.async_copy(u2_hbm.at[pl.ds(b, C)], u2v, sem)
            pltpu.async_copy(wc_hbm.at[pl.ds(cE + b, C)], wcv, sem)

        def wait_smalls(par):
            idxi_v, idxj_v, _, _, _, u0v, u1v, u2v, wcv, _, _, _ = bufs[par]
            sem = sem_sm[par]
            pltpu.make_async_copy(idxi_hbm.at[pl.ds(0, C)], idxi_v, sem).wait()
            pltpu.make_async_copy(idxj_hbm.at[pl.ds(0, C)], idxj_v, sem).wait()
            pltpu.make_async_copy(u0_hbm.at[pl.ds(0, C)], u0v, sem).wait()
            pltpu.make_async_copy(u1_hbm.at[pl.ds(0, C)], u1v, sem).wait()
            pltpu.make_async_copy(u2_hbm.at[pl.ds(0, C)], u2v, sem).wait()
            pltpu.make_async_copy(wc_hbm.at[pl.ds(0, C)], wcv, sem).wait()

        def issue_gathers(par):
            idxi_v, idxj_v, idxig, idxjg = bufs[par][0], bufs[par][1], bufs[par][2], bufs[par][3]
            mujv, muiv = bufs[par][9], bufs[par][10]
            sem = sem_g[par]
            idxjg[...] = idxj_v[...] + cN
            idxig[...] = idxi_v[...] + cN
            pltpu.async_copy(mu_hbm.at[idxjg], mujv, sem)
            pltpu.async_copy(mu_hbm.at[idxig], muiv, sem)

        def wait_gathers(par):
            mujv, muiv = bufs[par][9], bufs[par][10]
            sem = sem_g[par]
            pltpu.make_async_copy(mu_hbm.at[pl.ds(0, C)], mujv, sem).wait()
            pltpu.make_async_copy(mu_hbm.at[pl.ds(0, C)], muiv, sem).wait()

        def issue_scatter(par):
            idxi_v, scidx, stg = bufs[par][0], bufs[par][4], bufs[par][11]
            scidx[...] = idxi_v[...]
            pltpu.async_copy(stg, acc.at[scidx], sem_sc[par], add=True)

        def wait_scatter(par):
            stg = bufs[par][11]
            pltpu.make_async_copy(wc_hbm.at[pl.ds(0, C)], stg, sem_sc[par]).wait()

        def compute(par):
            u0v, u1v, u2v, wcv = bufs[par][5], bufs[par][6], bufs[par][7], bufs[par][8]
            mujv, muiv, stg = bufs[par][9], bufs[par][10], bufs[par][11]
            u0x = u0v[...]
            u1x = u1v[...]
            u2x = u2v[...]
            for j in range(C):
                b0 = lax.broadcast(u0x[j], (16,))
                b1 = lax.broadcast(u1x[j], (16,))
                b2 = lax.broadcast(u2x[j], (16,))
                for fc in range(FC):
                    o = fc * 16
                    mj0 = mujv[j, pl.ds(o, 16)]
                    mj1 = mujv[j, pl.ds(F + o, 16)]
                    mj2 = mujv[j, pl.ds(2 * F + o, 16)]
                    mi0 = muiv[j, pl.ds(o, 16)]
                    mi1 = muiv[j, pl.ds(F + o, 16)]
                    mi2 = muiv[j, pl.ds(2 * F + o, 16)]
                    w = wcv[j, pl.ds(o, 16)]
                    a = mi0 * mj0 + mi1 * mj1 + mi2 * mj2
                    pj = b0 * mj0 + b1 * mj1 + b2 * mj2
                    pi = b0 * mi0 + b1 * mi1 + b2 * mi2
                    stg[j, pl.ds(o, 16)] = w * (a - pi * pj)

        zv = jnp.zeros((16,), jnp.float32)
        stg0 = bufs[0][11]

        def zrow(i, carry):
            for fc in range(FC):
                stg0[i, pl.ds(fc * 16, 16)] = zv
            return carry
        lax.fori_loop(0, C, zrow, 0)
        for zz in range(NPT // C):
            pltpu.async_copy(stg0, acc.at[pl.ds(s * NPT + zz * C, C)], sem_sc[0])
        for zz in range(NPT // C):
            pltpu.make_async_copy(wc_hbm.at[pl.ds(0, C)],
                                  acc.at[pl.ds(s * NPT + zz * C, C)],
                                  sem_sc[0]).wait()
        plsc.subcore_barrier()

        issue_smalls(0, 0)
        issue_smalls(1, 1)
        wait_smalls(0)
        issue_gathers(0)

        def half(g, par):
            @pl.when(g + 1 < NCH)
            def _():
                wait_smalls(1 - par)
                issue_gathers(1 - par)
            wait_gathers(par)

            @pl.when(g >= 2)
            def _():
                wait_scatter(par)
            compute(par)
            issue_scatter(par)

            @pl.when(g + 2 < NCH)
            def _():
                issue_smalls(g + 2, par)

        def pair(gi, carry):
            g0 = gi * 2
            half(g0, 0)
            half(g0 + 1, 1)
            return carry
        lax.fori_loop(0, NCH // 2, pair, 0)

        wait_scatter(0)
        wait_scatter(1)
        plsc.subcore_barrier()
        pltpu.sync_copy(acc.at[pl.ds(s * NPT, NPT)],
                        out_hbm.at[pl.ds(c * NP + s * NPT, NPT)])

    return sc_stage, NP


def kernel(q, mu_electric_field, mu_magnetic_field, f_ij, d_ij, v_ij, idx_i, idx_j,
           rcut_ij, W1_e, b1_e, W2_e, b2_e, Wt_e, bt_e, W1_m, b1_m, W2_m, b2_m,
           Wt_m, bt_m):
    N, _, F = q.shape
    E, R = f_ij.shape

    EB = 640
    gb = E // EB
    W1s = jnp.stack([W1_e, W1_m])
    b1s = jnp.stack([b1_e, b1_m]).reshape(2, 1, F)
    W2s = jnp.stack([W2_e, W2_m])
    b2s = jnp.stack([b2_e, b2_m]).reshape(2, 1, F)
    wc_cat, u0, u1, u2 = pl.pallas_call(
        _edge_weights_body,
        grid=(2, gb),
        in_specs=[
            pl.BlockSpec((EB, R), lambda fi, i: (i, 0)),
            pl.BlockSpec((EB, 1), lambda fi, i: (i, 0)),
            pl.BlockSpec((EB, 1), lambda fi, i: (i, 0)),
            pl.BlockSpec((EB, 3), lambda fi, i: (i, 0)),
            pl.BlockSpec((1, R, F), lambda fi, i: (fi, 0, 0)),
            pl.BlockSpec((1, 1, F), lambda fi, i: (fi, 0, 0)),
            pl.BlockSpec((1, F, F), lambda fi, i: (fi, 0, 0)),
            pl.BlockSpec((1, 1, F), lambda fi, i: (fi, 0, 0)),
        ],
        out_specs=[
            pl.BlockSpec((EB, F), lambda fi, i, _g=gb: (fi * _g + i, 0)),
            pl.BlockSpec((EB, 1), lambda fi, i: (i, 0)),
            pl.BlockSpec((EB, 1), lambda fi, i: (i, 0)),
            pl.BlockSpec((EB, 1), lambda fi, i: (i, 0)),
        ],
        out_shape=[
            jax.ShapeDtypeStruct((2 * E, F), jnp.float32),
            jax.ShapeDtypeStruct((E, 1), jnp.float32),
            jax.ShapeDtypeStruct((E, 1), jnp.float32),
            jax.ShapeDtypeStruct((E, 1), jnp.float32),
        ],
    )(f_ij, d_ij.reshape(E, 1), rcut_ij.reshape(E, 1), v_ij, W1s, b1s, W2s, b2s)

    mu_cat = jnp.concatenate([mu_electric_field.reshape(N, 3 * F),
                              mu_magnetic_field.reshape(N, 3 * F)], axis=0)
    sc_stage, NP = _make_sc_stage(N, E, F)
    acc_cat = sc_stage(mu_cat, wc_cat, u0.reshape(E), u1.reshape(E), u2.reshape(E),
                       idx_i, idx_j)

    NB = 80
    grid_c = N // NB
    off_m = NP // NB
    full = lambda shape: pl.BlockSpec(shape, lambda i: (0, 0))
    out = pl.pallas_call(
        _node_transform_body,
        grid=(grid_c,),
        in_specs=[
            pl.BlockSpec((NB, F), lambda i: (i, 0)),
            pl.BlockSpec((NB, F), lambda i, _o=off_m: (_o + i, 0)),
            full((F, F)), full((1, F)),
            full((F, F)), full((1, F)),
        ],
        out_specs=pl.BlockSpec((NB, F), lambda i: (i, 0)),
        out_shape=jax.ShapeDtypeStruct((N, F), jnp.float32),
    )(acc_cat, acc_cat, Wt_e, bt_e.reshape(1, F), Wt_m, bt_m.reshape(1, F))

    return out.reshape(N, 1, F)


# speedup vs baseline: 1.2445x; 1.2445x over previous
import functools

import jax
import jax.numpy as jnp
from jax import lax
from jax.experimental import pallas as pl
from jax.experimental.pallas import tpu as pltpu
from jax.experimental.pallas import tpu_sc as plsc

_LOG2 = 0.6931471805599453
_SQRT3 = 1.7320508075688772


def _ssp(x):
    return jnp.maximum(x, 0.0) + jnp.log1p(jnp.exp(-jnp.abs(x))) - _LOG2


def _edge_weights_body(f_ref, d_ref, rc_ref, v_ref,
                       w1_ref, b1_ref, w2_ref, b2_ref,
                       wc_ref, u3_ref):
    f = f_ref[...]
    d = d_ref[...]
    rc = rc_ref[...]
    invd = 1.0 / d
    scale = rc * invd * invd * invd
    w1 = w1_ref[0]
    b1 = b1_ref[0]
    w2 = w2_ref[0]
    b2 = b2_ref[0]
    h = _ssp(jnp.dot(f, w1, preferred_element_type=jnp.float32) + b1)
    wc_ref[...] = (jnp.dot(h, w2, preferred_element_type=jnp.float32) + b2) * scale
    us = _SQRT3 * invd
    v = v_ref[...]
    eb = v.shape[0]
    u3_ref[...] = jnp.concatenate(
        [jnp.broadcast_to(v[:, k:k + 1] * us, (eb, 16)) for k in range(3)], axis=1)


def _node_transform_body(pe_ref, pm_ref, wte_ref, bte_ref, wtm_ref, btm_ref, out_ref):
    ye = _ssp(jnp.dot(pe_ref[...], wte_ref[...], preferred_element_type=jnp.float32) + bte_ref[...])
    ym = _ssp(jnp.dot(pm_ref[...], wtm_ref[...], preferred_element_type=jnp.float32) + btm_ref[...])
    out_ref[...] = ye + ym


def _make_sc_stage(N, E, F):
    NSUB = 16
    EPT = E // NSUB
    C = 16
    NCH = EPT // C
    NP = (N + NSUB * C - 1) // (NSUB * C) * (NSUB * C)
    NPT = NP // NSUB
    FC = F // 16
    TF = 3 * F

    mesh = plsc.VectorSubcoreMesh(core_axis_name="c", subcore_axis_name="s")

    D = 4
    scratch = [pltpu.VMEM_SHARED((NP, F), jnp.float32)]
    for _ in range(D):
        scratch += [
            pltpu.VMEM((C,), jnp.int32),
            pltpu.VMEM((C,), jnp.int32),
            pltpu.VMEM((C, 48), jnp.float32),
            pltpu.VMEM((C, F), jnp.float32),
        ]
    for _ in range(2):
        scratch += [
            pltpu.VMEM((C,), jnp.int32),
            pltpu.VMEM((C,), jnp.int32),
            pltpu.VMEM((C, TF), jnp.float32),
            pltpu.VMEM((C, TF), jnp.float32),
            pltpu.VMEM((C,), jnp.int32),
            pltpu.VMEM((C, F), jnp.float32),
        ]
    scratch += [pltpu.SemaphoreType.DMA] * (D + 2 + 2)

    @functools.partial(
        pl.kernel,
        out_type=jax.ShapeDtypeStruct((2 * NP, F), jnp.float32),
        mesh=mesh,
        scratch_types=scratch,
    )
    def sc_stage(mu_hbm, wc_hbm, u3_hbm, idxi_hbm, idxj_hbm,
                 out_hbm, *scr):
        acc = scr[0]
        bufs = tuple(scr[1 + i * 4: 1 + (i + 1) * 4] for i in range(D))
        gbase = 1 + 4 * D
        gbufs = tuple(scr[gbase + i * 6: gbase + (i + 1) * 6] for i in range(2))
        sem_sm = scr[gbase + 12: gbase + 12 + D]
        sem_g = scr[gbase + 12 + D: gbase + 12 + D + 2]
        sem_sc = scr[gbase + 12 + D + 2: gbase + 12 + D + 4]
        c = lax.axis_index("c")
        s = lax.axis_index("s")
        cN = c * N
        cE = c * E
        base0 = s * EPT

        def issue_smalls(g, par):
            idxi_v, idxj_v, u3v, wcv = bufs[par]
            sem = sem_sm[par]
            b = base0 + g * C
            pltpu.async_copy(idxi_hbm.at[pl.ds(b, C)], idxi_v, sem)
            pltpu.async_copy(idxj_hbm.at[pl.ds(b, C)], idxj_v, sem)
            pltpu.async_copy(u3_hbm.at[pl.ds(b, C)], u3v, sem)
            pltpu.async_copy(wc_hbm.at[pl.ds(cE + b, C)], wcv, sem)

        def wait_smalls(par):
            idxi_v, idxj_v, u3v, wcv = bufs[par]
            sem = sem_sm[par]
            pltpu.make_async_copy(idxi_hbm.at[pl.ds(0, C)], idxi_v, sem).wait()
            pltpu.make_async_copy(idxj_hbm.at[pl.ds(0, C)], idxj_v, sem).wait()
            pltpu.make_async_copy(u3_hbm.at[pl.ds(0, C)], u3v, sem).wait()
            pltpu.make_async_copy(wc_hbm.at[pl.ds(0, C)], wcv, sem).wait()

        def issue_gathers(par, gpar):
            idxi_v, idxj_v = bufs[par][0], bufs[par][1]
            idxig, idxjg, mujv, muiv = gbufs[gpar][0], gbufs[gpar][1], gbufs[gpar][2], gbufs[gpar][3]
            sem = sem_g[gpar]
            idxjg[...] = idxj_v[...] + cN
            idxig[...] = idxi_v[...] + cN
            pltpu.async_copy(mu_hbm.at[idxjg], mujv, sem)
            pltpu.async_copy(mu_hbm.at[idxig], muiv, sem)

        def wait_gathers(gpar):
            mujv, muiv = gbufs[gpar][2], gbufs[gpar][3]
            sem = sem_g[gpar]
            pltpu.make_async_copy(mu_hbm.at[pl.ds(0, C)], mujv, sem).wait()
            pltpu.make_async_copy(mu_hbm.at[pl.ds(0, C)], muiv, sem).wait()

        def issue_scatter(par, gpar):
            idxi_v = bufs[par][0]
            scidx, stg = gbufs[gpar][4], gbufs[gpar][5]
            scidx[...] = idxi_v[...]
            pltpu.async_copy(stg, acc.at[scidx], sem_sc[gpar], add=True)

        def wait_scatter(gpar):
            stg = gbufs[gpar][5]
            pltpu.make_async_copy(wc_hbm.at[pl.ds(0, C)], stg, sem_sc[gpar]).wait()

        def compute(par, gpar):
            u3v, wcv = bufs[par][2], bufs[par][3]
            mujv, muiv, stg = gbufs[gpar][2], gbufs[gpar][3], gbufs[gpar][5]

            def edge(j, carry):
                b0 = u3v[j, pl.ds(0, 16)]
                b1 = u3v[j, pl.ds(16, 16)]
                b2 = u3v[j, pl.ds(32, 16)]
                for fc in range(FC):
                    o = fc * 16
                    mj0 = mujv[j, pl.ds(o, 16)]
                    mj1 = mujv[j, pl.ds(F + o, 16)]
                    mj2 = mujv[j, pl.ds(2 * F + o, 16)]
                    mi0 = muiv[j, pl.ds(o, 16)]
                    mi1 = muiv[j, pl.ds(F + o, 16)]
                    mi2 = muiv[j, pl.ds(2 * F + o, 16)]
                    w = wcv[j, pl.ds(o, 16)]
                    a = mi0 * mj0 + mi1 * mj1 + mi2 * mj2
                    pj = b0 * mj0 + b1 * mj1 + b2 * mj2
                    pi = b0 * mi0 + b1 * mi1 + b2 * mi2
                    stg[j, pl.ds(o, 16)] = w * (a - pi * pj)
                return carry
            lax.fori_loop(0, C, edge, 0)

        zv = jnp.zeros((16,), jnp.float32)
        stg0 = gbufs[0][5]

        def zrow(i, carry):
            for fc in range(FC):
                stg0[i, pl.ds(fc * 16, 16)] = zv
            return carry
        lax.fori_loop(0, C, zrow, 0)
        for zz in range(NPT // C):
            pltpu.async_copy(stg0, acc.at[pl.ds(s * NPT + zz * C, C)], sem_sc[0])
        for zz in range(NPT // C):
            pltpu.make_async_copy(wc_hbm.at[pl.ds(0, C)],
                                  acc.at[pl.ds(s * NPT + zz * C, C)],
                                  sem_sc[0]).wait()
        plsc.subcore_barrier()

        issue_smalls(0, 0)
        issue_smalls(1, 1)
        issue_smalls(2, 2)
        issue_smalls(3, 3)
        wait_smalls(0)
        issue_gathers(0, 0)
        wait_smalls(1)
        issue_gathers(1, 1)

        def slot(g, par, gpar):
            wait_gathers(gpar)

            @pl.when(g >= 2)
            def _():
                wait_scatter(gpar)
            compute(par, gpar)
            issue_scatter(par, gpar)

            @pl.when(g + 2 < NCH)
            def _():
                wait_smalls((par + 2) % D)
                issue_gathers((par + 2) % D, gpar)

            @pl.when(g + 4 < NCH)
            def _():
                issue_smalls(g + 4, par)

        def four(gi, carry):
            g0 = gi * 4
            for j in range(4):
                g = g0 + j

                @pl.when(g < NCH)
                def _(_g=g, _j=j):
                    slot(_g, _j % D, _j % 2)
            return carry
        lax.fori_loop(0, (NCH + 3) // 4, four, 0)

        wait_scatter(0)
        wait_scatter(1)
        plsc.subcore_barrier()
        pltpu.sync_copy(acc.at[pl.ds(s * NPT, NPT)],
                        out_hbm.at[pl.ds(c * NP + s * NPT, NPT)])

    return sc_stage, NP


def kernel(q, mu_electric_field, mu_magnetic_field, f_ij, d_ij, v_ij, idx_i, idx_j,
           rcut_ij, W1_e, b1_e, W2_e, b2_e, Wt_e, bt_e, W1_m, b1_m, W2_m, b2_m,
           Wt_m, bt_m):
    N, _, F = q.shape
    E, R = f_ij.shape

    EB = 640
    gb = E // EB
    W1s = jnp.stack([W1_e, W1_m])
    b1s = jnp.stack([b1_e, b1_m]).reshape(2, 1, F)
    W2s = jnp.stack([W2_e, W2_m])
    b2s = jnp.stack([b2_e, b2_m]).reshape(2, 1, F)
    wc_cat, u3 = pl.pallas_call(
        _edge_weights_body,
        grid=(2, gb),
        in_specs=[
            pl.BlockSpec((EB, R), lambda fi, i: (i, 0)),
            pl.BlockSpec((EB, 1), lambda fi, i: (i, 0)),
            pl.BlockSpec((EB, 1), lambda fi, i: (i, 0)),
            pl.BlockSpec((EB, 3), lambda fi, i: (i, 0)),
            pl.BlockSpec((1, R, F), lambda fi, i: (fi, 0, 0)),
            pl.BlockSpec((1, 1, F), lambda fi, i: (fi, 0, 0)),
            pl.BlockSpec((1, F, F), lambda fi, i: (fi, 0, 0)),
            pl.BlockSpec((1, 1, F), lambda fi, i: (fi, 0, 0)),
        ],
        out_specs=[
            pl.BlockSpec((EB, F), lambda fi, i, _g=gb: (fi * _g + i, 0)),
            pl.BlockSpec((EB, 48), lambda fi, i: (i, 0)),
        ],
        out_shape=[
            jax.ShapeDtypeStruct((2 * E, F), jnp.float32),
            jax.ShapeDtypeStruct((E, 48), jnp.float32),
        ],
    )(f_ij, d_ij.reshape(E, 1), rcut_ij.reshape(E, 1), v_ij, W1s, b1s, W2s, b2s)

    mu_cat = jnp.concatenate([mu_electric_field.reshape(N, 3 * F),
                              mu_magnetic_field.reshape(N, 3 * F)], axis=0)
    sc_stage, NP = _make_sc_stage(N, E, F)
    acc_cat = sc_stage(mu_cat, wc_cat, u3, idx_i, idx_j)

    NB = 80
    grid_c = N // NB
    off_m = NP // NB
    full = lambda shape: pl.BlockSpec(shape, lambda i: (0, 0))
    out = pl.pallas_call(
        _node_transform_body,
        grid=(grid_c,),
        in_specs=[
            pl.BlockSpec((NB, F), lambda i: (i, 0)),
            pl.BlockSpec((NB, F), lambda i, _o=off_m: (_o + i, 0)),
            full((F, F)), full((1, F)),
            full((F, F)), full((1, F)),
        ],
        out_specs=pl.BlockSpec((NB, F), lambda i: (i, 0)),
        out_shape=jax.ShapeDtypeStruct((N, F), jnp.float32),
    )(acc_cat, acc_cat, Wt_e, bt_e.reshape(1, F), Wt_m, bt_m.reshape(1, F))

    return out.reshape(N, 1, F)
